# baseline (device time: 68430 ns/iter reference)
import jax
import jax.numpy as jnp
from jax import lax
from jax.experimental import pallas as pl
from jax.experimental.pallas import tpu as pltpu

N = 4

_sem_signal = getattr(pl, "semaphore_signal", None) or pltpu.semaphore_signal
_sem_wait = getattr(pl, "semaphore_wait", None) or pltpu.semaphore_wait
_DevIdType = getattr(pl, "DeviceIdType", None) or pltpu.DeviceIdType
_CompilerParams = getattr(pltpu, "CompilerParams", None) or pltpu.TPUCompilerParams


def kernel(x, assign, W1, W2):
    T, D = x.shape
    E, _, F = W1.shape
    a2 = assign.reshape(T, 1)

    def body(x_ref, a_ref, w1_ref, w2_ref, out_ref,
             xg, ag, acc, rsb, w1b, w2b,
             sx_send, sx_recv, sa_send, sa_recv, rs_send, rs_recv):
        my_x = lax.axis_index("x")
        my_y = lax.axis_index("y")
        my_z = lax.axis_index("z")
        left = (my_z - 1) % N
        right = (my_z + 1) % N

        barrier = pltpu.get_barrier_semaphore()
        for nz in (left, right):
            _sem_signal(barrier, inc=1, device_id=(my_x, my_y, nz),
                        device_id_type=_DevIdType.MESH)
        _sem_wait(barrier, 2)

        w1b[...] = w1_ref[...].astype(jnp.bfloat16)
        w2b[...] = w2_ref[...].astype(jnp.bfloat16)

        xg[my_z] = x_ref[...].astype(jnp.bfloat16)
        ag[my_z] = a_ref[...]

        def compute_chunk(c):
            xc = xg[c]
            a = ag[c]
            r = jnp.zeros((T, D), jnp.float32)
            for e in range(E):
                eg = my_z * E + e
                xm = jnp.where(a == eg, xc, 0)
                h = jnp.maximum(
                    jnp.dot(xm, w1b[e], preferred_element_type=jnp.float32),
                    0.0,
                ).astype(jnp.bfloat16)
                r = r + jnp.dot(h, w2b[e], preferred_element_type=jnp.float32)
            acc[c] = r.astype(jnp.bfloat16)

        for h in range(N - 1):
            cs = (my_z - h) % N
            rx = pltpu.make_async_remote_copy(
                src_ref=xg.at[cs], dst_ref=xg.at[cs],
                send_sem=sx_send.at[h], recv_sem=sx_recv.at[h],
                device_id=(my_x, my_y, right),
                device_id_type=_DevIdType.MESH,
            )
            ra = pltpu.make_async_remote_copy(
                src_ref=ag.at[cs], dst_ref=ag.at[cs],
                send_sem=sa_send.at[h], recv_sem=sa_recv.at[h],
                device_id=(my_x, my_y, right),
                device_id_type=_DevIdType.MESH,
            )
            rx.start()
            ra.start()
            compute_chunk(cs)
            rx.wait()
            ra.wait()
        compute_chunk((my_z + 1) % N)

        for s in range(N - 1):
            cs = (my_z - s - 1) % N
            rr = pltpu.make_async_remote_copy(
                src_ref=acc.at[cs], dst_ref=rsb.at[s],
                send_sem=rs_send.at[s], recv_sem=rs_recv.at[s],
                device_id=(my_x, my_y, right),
                device_id_type=_DevIdType.MESH,
            )
            rr.start()
            rr.wait()
            cr = (my_z - s - 2) % N
            acc[cr] = acc[cr] + rsb[s]

        out_ref[...] = acc[my_z].astype(jnp.float32)

    return pl.pallas_call(
        body,
        out_shape=jax.ShapeDtypeStruct((T, D), jnp.float32),
        in_specs=[pl.BlockSpec(memory_space=pltpu.VMEM)] * 4,
        out_specs=pl.BlockSpec(memory_space=pltpu.VMEM),
        scratch_shapes=[
            pltpu.VMEM((N, T, D), jnp.bfloat16),
            pltpu.VMEM((N, T, 1), jnp.int32),
            pltpu.VMEM((N, T, D), jnp.bfloat16),
            pltpu.VMEM((N - 1, T, D), jnp.bfloat16),
            pltpu.VMEM((E, D, F), jnp.bfloat16),
            pltpu.VMEM((E, F, D), jnp.bfloat16),
            pltpu.SemaphoreType.DMA((N - 1,)),
            pltpu.SemaphoreType.DMA((N - 1,)),
            pltpu.SemaphoreType.DMA((N - 1,)),
            pltpu.SemaphoreType.DMA((N - 1,)),
            pltpu.SemaphoreType.DMA((N - 1,)),
            pltpu.SemaphoreType.DMA((N - 1,)),
        ],
        compiler_params=_CompilerParams(collective_id=0),
    )(x, a2, W1, W2)


# device time: 59419 ns/iter; 1.1517x vs baseline; 1.1517x over previous
import jax
import jax.numpy as jnp
from jax import lax
from jax.experimental import pallas as pl
from jax.experimental.pallas import tpu as pltpu

N = 4

_sem_signal = getattr(pl, "semaphore_signal", None) or pltpu.semaphore_signal
_sem_wait = getattr(pl, "semaphore_wait", None) or pltpu.semaphore_wait
_DevIdType = getattr(pl, "DeviceIdType", None) or pltpu.DeviceIdType
_CompilerParams = getattr(pltpu, "CompilerParams", None) or pltpu.TPUCompilerParams


def kernel(x, assign, W1, W2):
    T, D = x.shape
    E, _, F = W1.shape
    a2 = assign.reshape(T, 1)

    def body(x_ref, a_ref, w1_ref, w2_ref, out_ref,
             xg, ag, acc, rsb, w1b, w2b,
             sx_send, sx_recv, sa_send, sa_recv, rs_send, rs_recv):
        my_x = lax.axis_index("x")
        my_y = lax.axis_index("y")
        my_z = lax.axis_index("z")
        left = (my_z - 1) % N
        right = (my_z + 1) % N

        barrier = pltpu.get_barrier_semaphore()
        for nz in (left, right):
            _sem_signal(barrier, inc=1, device_id=(my_x, my_y, nz),
                        device_id_type=_DevIdType.MESH)
        _sem_wait(barrier, 2)

        w1b[...] = w1_ref[...].astype(jnp.bfloat16)
        w2b[...] = w2_ref[...].astype(jnp.bfloat16)

        xg[my_z] = x_ref[...].astype(jnp.bfloat16)
        ag[my_z] = a_ref[...]

        def compute_chunk(c):
            xc = xg[c]
            a = ag[c]
            r = jnp.zeros((T, D), jnp.float32)
            for e in range(E):
                eg = my_z * E + e
                xm = jnp.where(a == eg, xc, 0)
                h = jnp.maximum(
                    jnp.dot(xm, w1b[e], preferred_element_type=jnp.float32),
                    0.0,
                ).astype(jnp.bfloat16)
                r = r + jnp.dot(h, w2b[e], preferred_element_type=jnp.float32)
            acc[c] = r.astype(jnp.bfloat16)

        def ag_rdmas(h):
            cs = (my_z - h) % N
            rx = pltpu.make_async_remote_copy(
                src_ref=xg.at[cs], dst_ref=xg.at[cs],
                send_sem=sx_send.at[h], recv_sem=sx_recv.at[h],
                device_id=(my_x, my_y, right),
                device_id_type=_DevIdType.MESH,
            )
            ra = pltpu.make_async_remote_copy(
                src_ref=ag.at[cs], dst_ref=ag.at[cs],
                send_sem=sa_send.at[h], recv_sem=sa_recv.at[h],
                device_id=(my_x, my_y, right),
                device_id_type=_DevIdType.MESH,
            )
            return rx, ra

        def rs_rdma(s):
            cs = (my_z - s - 1) % N
            return pltpu.make_async_remote_copy(
                src_ref=acc.at[cs], dst_ref=rsb.at[s],
                send_sem=rs_send.at[s], recv_sem=rs_recv.at[s],
                device_id=(my_x, my_y, right),
                device_id_type=_DevIdType.MESH,
            )

        rx0, ra0 = ag_rdmas(0)
        rx0.start(); ra0.start()
        compute_chunk(my_z)
        rx0.wait_recv(); ra0.wait_recv()

        rx1, ra1 = ag_rdmas(1)
        rx1.start(); ra1.start()
        compute_chunk((my_z - 1) % N)
        rr0 = rs_rdma(0)
        rr0.start()
        rx1.wait_recv(); ra1.wait_recv()
        rx0.wait_send(); ra0.wait_send()

        rx2, ra2 = ag_rdmas(2)
        rx2.start(); ra2.start()
        compute_chunk((my_z - 2) % N)
        rr0.wait_recv()
        acc[(my_z - 2) % N] = acc[(my_z - 2) % N] + rsb[0]
        rr1 = rs_rdma(1)
        rr1.start()
        rx2.wait_recv(); ra2.wait_recv()
        rx1.wait_send(); ra1.wait_send()

        compute_chunk((my_z + 1) % N)
        rr1.wait_recv()
        acc[(my_z + 1) % N] = acc[(my_z + 1) % N] + rsb[1]
        rr2 = rs_rdma(2)
        rr2.start()
        rr2.wait_recv()
        out_ref[...] = (acc[my_z] + rsb[2]).astype(jnp.float32)

        rx2.wait_send(); ra2.wait_send()
        rr0.wait_send(); rr1.wait_send(); rr2.wait_send()

    return pl.pallas_call(
        body,
        out_shape=jax.ShapeDtypeStruct((T, D), jnp.float32),
        in_specs=[pl.BlockSpec(memory_space=pltpu.VMEM)] * 4,
        out_specs=pl.BlockSpec(memory_space=pltpu.VMEM),
        scratch_shapes=[
            pltpu.VMEM((N, T, D), jnp.bfloat16),
            pltpu.VMEM((N, T, 1), jnp.int32),
            pltpu.VMEM((N, T, D), jnp.bfloat16),
            pltpu.VMEM((N - 1, T, D), jnp.bfloat16),
            pltpu.VMEM((E, D, F), jnp.bfloat16),
            pltpu.VMEM((E, F, D), jnp.bfloat16),
            pltpu.SemaphoreType.DMA((N - 1,)),
            pltpu.SemaphoreType.DMA((N - 1,)),
            pltpu.SemaphoreType.DMA((N - 1,)),
            pltpu.SemaphoreType.DMA((N - 1,)),
            pltpu.SemaphoreType.DMA((N - 1,)),
            pltpu.SemaphoreType.DMA((N - 1,)),
        ],
        compiler_params=_CompilerParams(collective_id=0),
    )(x, a2, W1, W2)


# device time: 32001 ns/iter; 2.1384x vs baseline; 1.8568x over previous
import jax
import jax.numpy as jnp
from jax import lax
from jax.experimental import pallas as pl
from jax.experimental.pallas import tpu as pltpu

N = 4
C = 160

_sem_signal = getattr(pl, "semaphore_signal", None) or pltpu.semaphore_signal
_sem_wait = getattr(pl, "semaphore_wait", None) or pltpu.semaphore_wait
_DevIdType = getattr(pl, "DeviceIdType", None) or pltpu.DeviceIdType
_CompilerParams = getattr(pltpu, "CompilerParams", None) or pltpu.TPUCompilerParams


def kernel(x, assign, W1, W2):
    T, D = x.shape
    E, _, F = W1.shape
    a_col = assign.reshape(T, 1)
    a_row = assign.reshape(1, T)

    def body(x_ref, ac_ref, ar_ref, w1_ref, w2_ref, out_ref,
             cx, ca, xin, ain, rres, rb, w1b, w2b,
             s1x_send, s1x_recv, s1a_send, s1a_recv, s3_send, s3_recv):
        my_x = lax.axis_index("x")
        my_y = lax.axis_index("y")
        my_z = lax.axis_index("z")
        peers = [i + (i >= my_z).astype(jnp.int32) for i in range(N - 1)]

        barrier = pltpu.get_barrier_semaphore()
        for p in peers:
            _sem_signal(barrier, inc=1, device_id=(my_x, my_y, p),
                        device_id_type=_DevIdType.MESH)
        _sem_wait(barrier, N - 1)

        w1b[...] = w1_ref[...].astype(jnp.bfloat16)
        w2b[...] = w2_ref[...].astype(jnp.bfloat16)
        xb = x_ref[...].astype(jnp.bfloat16)
        af1 = (ac_ref[...] + 1).astype(jnp.bfloat16)
        owner = ar_ref[...] >> 1
        iota_c = lax.broadcasted_iota(jnp.int32, (C, T), 0)
        tri = (lax.broadcasted_iota(jnp.int32, (T, T), 0)
               <= lax.broadcasted_iota(jnp.int32, (T, T), 1)).astype(jnp.bfloat16)

        S = []
        sends = []
        for i in range(N - 1):
            p = peers[i]
            m = owner == p
            rank = jnp.dot(m.astype(jnp.bfloat16), tri,
                           preferred_element_type=jnp.float32
                           ).astype(jnp.int32) - 1
            Si = ((iota_c == rank) & m).astype(jnp.bfloat16)
            S.append(Si)
            cx[i] = jnp.dot(Si, xb,
                            preferred_element_type=jnp.float32).astype(jnp.bfloat16)
            ca[i] = jnp.dot(Si, af1,
                            preferred_element_type=jnp.float32).astype(jnp.bfloat16)
            j = my_z - (my_z > p).astype(jnp.int32)
            rx = pltpu.make_async_remote_copy(
                src_ref=cx.at[i], dst_ref=xin.at[pl.ds(C * j, C)],
                send_sem=s1x_send.at[i], recv_sem=s1x_recv.at[j],
                device_id=(my_x, my_y, p), device_id_type=_DevIdType.MESH,
            )
            ra = pltpu.make_async_remote_copy(
                src_ref=ca.at[i], dst_ref=ain.at[pl.ds(C * j, C)],
                send_sem=s1a_send.at[i], recv_sem=s1a_recv.at[j],
                device_id=(my_x, my_y, p), device_id_type=_DevIdType.MESH,
            )
            rx.start()
            ra.start()
            sends += [rx, ra]

        def ffn(xv, av):
            r = jnp.zeros((xv.shape[0], D), jnp.float32)
            for e in range(E):
                eg1 = (my_z * E + e + 1).astype(jnp.bfloat16)
                xm = jnp.where(av == eg1, xv, 0)
                h = jnp.maximum(
                    jnp.dot(xm, w1b[e], preferred_element_type=jnp.float32),
                    0.0,
                ).astype(jnp.bfloat16)
                r = r + jnp.dot(h, w2b[e], preferred_element_type=jnp.float32)
            return r

        r_own = ffn(xb, af1)

        for i in range(N - 1):
            for buf, slot_shape, sems in ((xin, C, (s1x_send, s1x_recv)),
                                          (ain, C, (s1a_send, s1a_recv))):
                d = pltpu.make_async_remote_copy(
                    src_ref=cx.at[i] if buf is xin else ca.at[i],
                    dst_ref=buf.at[pl.ds(C * i, C)],
                    send_sem=sems[0].at[i], recv_sem=sems[1].at[i],
                    device_id=(my_x, my_y, my_z),
                    device_id_type=_DevIdType.MESH,
                )
                d.wait_recv()
        rres[...] = ffn(xin[...], ain[...]).astype(jnp.bfloat16)

        for i in range(N - 1):
            p = peers[i]
            j = my_z - (my_z > p).astype(jnp.int32)
            r3 = pltpu.make_async_remote_copy(
                src_ref=rres.at[pl.ds(C * i, C)], dst_ref=rb.at[j],
                send_sem=s3_send.at[i], recv_sem=s3_recv.at[j],
                device_id=(my_x, my_y, p), device_id_type=_DevIdType.MESH,
            )
            r3.start()
            sends.append(r3)

        out_val = r_own
        for i in range(N - 1):
            d3 = pltpu.make_async_remote_copy(
                src_ref=rres.at[pl.ds(C * i, C)], dst_ref=rb.at[i],
                send_sem=s3_send.at[i], recv_sem=s3_recv.at[i],
                device_id=(my_x, my_y, my_z), device_id_type=_DevIdType.MESH,
            )
            d3.wait_recv()
            out_val = out_val + lax.dot_general(
                S[i], rb[i], (((0,), (0,)), ((), ())),
                preferred_element_type=jnp.float32,
            )
        out_ref[...] = out_val

        for r in sends:
            r.wait_send()

    return pl.pallas_call(
        body,
        out_shape=jax.ShapeDtypeStruct((T, D), jnp.float32),
        in_specs=[pl.BlockSpec(memory_space=pltpu.VMEM)] * 5,
        out_specs=pl.BlockSpec(memory_space=pltpu.VMEM),
        scratch_shapes=[
            pltpu.VMEM((N - 1, C, D), jnp.bfloat16),
            pltpu.VMEM((N - 1, C, 1), jnp.bfloat16),
            pltpu.VMEM(((N - 1) * C, D), jnp.bfloat16),
            pltpu.VMEM(((N - 1) * C, 1), jnp.bfloat16),
            pltpu.VMEM(((N - 1) * C, D), jnp.bfloat16),
            pltpu.VMEM((N - 1, C, D), jnp.bfloat16),
            pltpu.VMEM((E, D, F), jnp.bfloat16),
            pltpu.VMEM((E, F, D), jnp.bfloat16),
            pltpu.SemaphoreType.DMA((N - 1,)),
            pltpu.SemaphoreType.DMA((N - 1,)),
            pltpu.SemaphoreType.DMA((N - 1,)),
            pltpu.SemaphoreType.DMA((N - 1,)),
            pltpu.SemaphoreType.DMA((N - 1,)),
            pltpu.SemaphoreType.DMA((N - 1,)),
        ],
        compiler_params=_CompilerParams(collective_id=0),
    )(x, a_col, a_row, W1, W2)


# device time: 28302 ns/iter; 2.4179x vs baseline; 1.1307x over previous
import jax
import jax.numpy as jnp
from jax import lax
from jax.experimental import pallas as pl
from jax.experimental.pallas import tpu as pltpu

N = 4
C = 160

_sem_signal = getattr(pl, "semaphore_signal", None) or pltpu.semaphore_signal
_sem_wait = getattr(pl, "semaphore_wait", None) or pltpu.semaphore_wait
_DevIdType = getattr(pl, "DeviceIdType", None) or pltpu.DeviceIdType
_CompilerParams = getattr(pltpu, "CompilerParams", None) or pltpu.TPUCompilerParams


def kernel(x, assign, W1, W2):
    T, D = x.shape
    E, _, F = W1.shape
    a_col = assign.reshape(T, 1)
    a_row = assign.reshape(1, T)

    def body(x_ref, ac_ref, ar_ref, w1_ref, w2_ref, out_ref,
             cx, ca, xin, ain, rres, rb, w1b, w2b,
             s1x_send, s1x_recv, s1a_send, s1a_recv, s3_send, s3_recv):
        my_x = lax.axis_index("x")
        my_y = lax.axis_index("y")
        my_z = lax.axis_index("z")
        peers = [i + (i >= my_z).astype(jnp.int32) for i in range(N - 1)]

        barrier = pltpu.get_barrier_semaphore()
        for p in peers:
            _sem_signal(barrier, inc=1, device_id=(my_x, my_y, p),
                        device_id_type=_DevIdType.MESH)
        _sem_wait(barrier, N - 1)

        xb = x_ref[...].astype(jnp.bfloat16)
        af1 = (ac_ref[...] + 1).astype(jnp.bfloat16)
        owner = ar_ref[...] >> 1
        iota_c = lax.broadcasted_iota(jnp.int32, (C, T), 0)
        tri = (lax.broadcasted_iota(jnp.int32, (T, T), 0)
               <= lax.broadcasted_iota(jnp.int32, (T, T), 1)).astype(jnp.bfloat16)

        S = []
        sends = []
        for i in range(N - 1):
            p = peers[i]
            m = owner == p
            rank = jnp.dot(m.astype(jnp.bfloat16), tri,
                           preferred_element_type=jnp.float32
                           ).astype(jnp.int32) - 1
            Si = ((iota_c == rank) & m).astype(jnp.bfloat16)
            S.append(Si)
            cx[i] = jnp.dot(Si, xb,
                            preferred_element_type=jnp.float32).astype(jnp.bfloat16)
            ca[i] = jnp.dot(Si, af1,
                            preferred_element_type=jnp.float32).astype(jnp.bfloat16)
            j = my_z - (my_z > p).astype(jnp.int32)
            rx = pltpu.make_async_remote_copy(
                src_ref=cx.at[i], dst_ref=xin.at[pl.ds(C * j, C)],
                send_sem=s1x_send.at[i], recv_sem=s1x_recv.at[j],
                device_id=(my_x, my_y, p), device_id_type=_DevIdType.MESH,
            )
            ra = pltpu.make_async_remote_copy(
                src_ref=ca.at[i], dst_ref=ain.at[pl.ds(C * j, C)],
                send_sem=s1a_send.at[i], recv_sem=s1a_recv.at[j],
                device_id=(my_x, my_y, p), device_id_type=_DevIdType.MESH,
            )
            rx.start()
            ra.start()
            sends += [rx, ra]

        w1b[...] = w1_ref[...].astype(jnp.bfloat16)
        w2b[...] = w2_ref[...].astype(jnp.bfloat16)

        def ffn(xv, av):
            r = jnp.zeros((xv.shape[0], D), jnp.float32)
            for e in range(E):
                eg1 = (my_z * E + e + 1).astype(jnp.bfloat16)
                xm = jnp.where(av == eg1, xv, 0)
                h = jnp.maximum(
                    jnp.dot(xm, w1b[e], preferred_element_type=jnp.float32),
                    0.0,
                ).astype(jnp.bfloat16)
                r = r + jnp.dot(h, w2b[e], preferred_element_type=jnp.float32)
            return r

        r_own = ffn(xb, af1)

        for i in range(N - 1):
            for src, buf, sems in ((cx, xin, (s1x_send, s1x_recv)),
                                   (ca, ain, (s1a_send, s1a_recv))):
                d = pltpu.make_async_remote_copy(
                    src_ref=src.at[i],
                    dst_ref=buf.at[pl.ds(C * i, C)],
                    send_sem=sems[0].at[i], recv_sem=sems[1].at[i],
                    device_id=(my_x, my_y, my_z),
                    device_id_type=_DevIdType.MESH,
                )
                d.wait_recv()
            rres[C * i:C * (i + 1)] = ffn(
                xin[C * i:C * (i + 1)], ain[C * i:C * (i + 1)]
            ).astype(jnp.bfloat16)
            p = peers[i]
            j = my_z - (my_z > p).astype(jnp.int32)
            r3 = pltpu.make_async_remote_copy(
                src_ref=rres.at[pl.ds(C * i, C)], dst_ref=rb.at[j],
                send_sem=s3_send.at[i], recv_sem=s3_recv.at[j],
                device_id=(my_x, my_y, p), device_id_type=_DevIdType.MESH,
            )
            r3.start()
            sends.append(r3)

        out_val = r_own
        for i in range(N - 1):
            d3 = pltpu.make_async_remote_copy(
                src_ref=rres.at[pl.ds(C * i, C)], dst_ref=rb.at[i],
                send_sem=s3_send.at[i], recv_sem=s3_recv.at[i],
                device_id=(my_x, my_y, my_z), device_id_type=_DevIdType.MESH,
            )
            d3.wait_recv()
            out_val = out_val + lax.dot_general(
                S[i], rb[i], (((0,), (0,)), ((), ())),
                preferred_element_type=jnp.float32,
            )
        out_ref[...] = out_val

        for r in sends:
            r.wait_send()

    return pl.pallas_call(
        body,
        out_shape=jax.ShapeDtypeStruct((T, D), jnp.float32),
        in_specs=[pl.BlockSpec(memory_space=pltpu.VMEM)] * 5,
        out_specs=pl.BlockSpec(memory_space=pltpu.VMEM),
        scratch_shapes=[
            pltpu.VMEM((N - 1, C, D), jnp.bfloat16),
            pltpu.VMEM((N - 1, C, 1), jnp.bfloat16),
            pltpu.VMEM(((N - 1) * C, D), jnp.bfloat16),
            pltpu.VMEM(((N - 1) * C, 1), jnp.bfloat16),
            pltpu.VMEM(((N - 1) * C, D), jnp.bfloat16),
            pltpu.VMEM((N - 1, C, D), jnp.bfloat16),
            pltpu.VMEM((E, D, F), jnp.bfloat16),
            pltpu.VMEM((E, F, D), jnp.bfloat16),
            pltpu.SemaphoreType.DMA((N - 1,)),
            pltpu.SemaphoreType.DMA((N - 1,)),
            pltpu.SemaphoreType.DMA((N - 1,)),
            pltpu.SemaphoreType.DMA((N - 1,)),
            pltpu.SemaphoreType.DMA((N - 1,)),
            pltpu.SemaphoreType.DMA((N - 1,)),
        ],
        compiler_params=_CompilerParams(collective_id=0),
    )(x, a_col, a_row, W1, W2)
